# Initial kernel scaffold; baseline (speedup 1.0000x reference)
import jax
import jax.numpy as jnp
from jax.experimental import pallas as pl


def kernel(input, adj, W):
    raise NotImplementedError("write your pallas kernel here")


# speedup vs baseline: 1.7671x; 1.7671x over previous
import functools

import jax
import jax.numpy as jnp
from jax.experimental import pallas as pl
from jax.experimental.pallas import tpu as pltpu

N = 4096
D_IN = 512
D_OUT = 512

BM = 512
BI = 256


def _linear_kernel(x_ref, w_ref, h_ref):
    h_ref[...] = jnp.dot(
        x_ref[...], w_ref[...], preferred_element_type=jnp.float32
    ).astype(jnp.bfloat16)


def _aggr_kernel(adj_ref, h_ref, o_ref):
    a = adj_ref[...]
    deg = jnp.sum(a, axis=1, keepdims=True)
    acc = jnp.dot(
        a.astype(jnp.bfloat16), h_ref[...], preferred_element_type=jnp.float32
    )
    o_ref[...] = acc / deg


@jax.jit
def kernel(input, adj, W):
    h = pl.pallas_call(
        _linear_kernel,
        grid=(N // BM,),
        in_specs=[
            pl.BlockSpec((BM, D_IN), lambda i: (i, 0)),
            pl.BlockSpec((D_IN, D_OUT), lambda i: (0, 0)),
        ],
        out_specs=pl.BlockSpec((BM, D_OUT), lambda i: (i, 0)),
        out_shape=jax.ShapeDtypeStruct((N, D_OUT), jnp.bfloat16),
        compiler_params=pltpu.CompilerParams(
            dimension_semantics=("arbitrary",),
        ),
    )(input, W)

    aggr = pl.pallas_call(
        _aggr_kernel,
        grid=(N // BI,),
        in_specs=[
            pl.BlockSpec((BI, N), lambda i: (i, 0)),
            pl.BlockSpec((N, D_OUT), lambda i: (0, 0)),
        ],
        out_specs=pl.BlockSpec((BI, D_OUT), lambda i: (i, 0)),
        out_shape=jax.ShapeDtypeStruct((N, D_OUT), jnp.float32),
        compiler_params=pltpu.CompilerParams(
            dimension_semantics=("arbitrary",),
        ),
    )(adj, h)

    return aggr


# speedup vs baseline: 1.9569x; 1.1074x over previous
import functools

import jax
import jax.numpy as jnp
from jax.experimental import pallas as pl
from jax.experimental.pallas import tpu as pltpu

N = 4096
D_IN = 512
D_OUT = 512

BM = 512
BI = 512


def _linear_kernel(x_ref, w_ref, h_ref):
    h_ref[...] = jnp.dot(
        x_ref[...], w_ref[...], preferred_element_type=jnp.float32
    ).astype(jnp.bfloat16)


def _aggr_kernel(adj_ref, h_ref, o_ref):
    a = adj_ref[...]
    deg = jnp.sum(a, axis=1, keepdims=True)
    acc = jnp.dot(
        a.astype(jnp.bfloat16), h_ref[...], preferred_element_type=jnp.float32
    )
    o_ref[...] = acc / deg


@jax.jit
def kernel(input, adj, W):
    h = pl.pallas_call(
        _linear_kernel,
        grid=(N // BM,),
        in_specs=[
            pl.BlockSpec((BM, D_IN), lambda i: (i, 0)),
            pl.BlockSpec((D_IN, D_OUT), lambda i: (0, 0)),
        ],
        out_specs=pl.BlockSpec((BM, D_OUT), lambda i: (i, 0)),
        out_shape=jax.ShapeDtypeStruct((N, D_OUT), jnp.bfloat16),
        compiler_params=pltpu.CompilerParams(
            dimension_semantics=("arbitrary",),
        ),
    )(input, W)

    aggr = pl.pallas_call(
        _aggr_kernel,
        grid=(N // BI,),
        in_specs=[
            pl.BlockSpec((BI, N), lambda i: (i, 0)),
            pl.BlockSpec((N, D_OUT), lambda i: (0, 0)),
        ],
        out_specs=pl.BlockSpec((BI, D_OUT), lambda i: (i, 0)),
        out_shape=jax.ShapeDtypeStruct((N, D_OUT), jnp.float32),
        compiler_params=pltpu.CompilerParams(
            dimension_semantics=("arbitrary",),
        ),
    )(adj, h)

    return aggr
